# baseline (speedup 1.0000x reference)
import functools

import jax
import jax.numpy as jnp
from jax import lax
from jax.experimental import pallas as pl
from jax.experimental.pallas import tpu as pltpu
from jax.experimental.pallas import tpu_sc as plsc

N = 10000
NPAD = 10240
D = 128
E = 320000
C = 64
NW = 32
NCHUNK = 5120
EPAD = NCHUNK * C - E
FULL_T = NCHUNK // NW
ROWS_PER_TILE = NPAD // 16
TSEG = 20
NSEG = FULL_T // TSEG
TBUF = TSEG * C
NSLOT = 5
DG = 3


def _sc_body(x_hbm, src_hbm, dst_hbm, w_hbm, part_hbm,
             src_all, dst_all, w_all,
             rows0, rows1, rows2, rows3, rows4,
             dstc0, dstc1, dstc2, dstc3, dstc4, acc_sh,
             gsem0, gsem1, gsem2, gsem3, gsem4,
             ssem0, ssem1, ssem2, ssem3, ssem4):
  rows = [rows0, rows1, rows2, rows3, rows4]
  dstc = [dstc0, dstc1, dstc2, dstc3, dstc4]
  gsem = [gsem0, gsem1, gsem2, gsem3, gsem4]
  ssem = [ssem0, ssem1, ssem2, ssem3, ssem4]
  c = lax.axis_index("c")
  s = lax.axis_index("s")
  wid = s * 2 + c
  base = wid * FULL_T * C

  def zrow(i, carry):
    for j in range(8):
      rows0[i, pl.ds(j * 16, 16)] = jnp.zeros((16,), jnp.float32)
    return carry

  lax.fori_loop(0, C, zrow, 0)
  zd = []
  for r in range(ROWS_PER_TILE // C):
    zd.append(pltpu.async_copy(
        rows0, acc_sh.at[pl.ds(s * ROWS_PER_TILE + r * C, C)], ssem0))
  for dsc in zd:
    dsc.wait()

  plsc.subcore_barrier()

  def scale_chunk(buf, k):
    def scale16(i, carry):
      wv = w_all[pl.ds(k * C + i * 16, 16)]
      for r in range(16):
        wb = jnp.broadcast_to(wv[r], (16,))
        row = i * 16 + r
        for j in range(8):
          sl = pl.ds(j * 16, 16)
          buf[row, sl] = buf[row, sl] * wb
      return carry

    lax.fori_loop(0, C // 16, scale16, 0)

  def stage_dst(b, k):
    for j in range(C // 16):
      dstc[b][pl.ds(j * 16, 16)] = dst_all[pl.ds(k * C + j * 16, 16)]

  def start_gather(k, b):
    pltpu.async_copy(x_hbm.at[src_all.at[pl.ds(k * C, C)]], rows[b], gsem[b])

  def wait_gather(k, b):
    pltpu.make_async_copy(x_hbm.at[src_all.at[pl.ds(k * C, C)]], rows[b],
                          gsem[b]).wait()

  def wait_scatter(b):
    pltpu.make_async_copy(rows[b], acc_sh.at[dstc[b]], ssem[b]).wait()

  def seg_body(g, carry):
    sbase = base + g * TBUF
    pltpu.sync_copy(src_hbm.at[pl.ds(sbase, TBUF)], src_all)
    pltpu.sync_copy(dst_hbm.at[pl.ds(sbase, TBUF)], dst_all)
    pltpu.sync_copy(w_hbm.at[pl.ds(sbase, TBUF)], w_all)
    for j in range(DG):
      start_gather(j, j)

    def iter_body(t, carry2):
      for b in range(NSLOT):
        k = NSLOT * t + b
        wait_gather(k, b)
        scale_chunk(rows[b], k)
        stage_dst(b, k)
        pltpu.async_copy(rows[b], acc_sh.at[dstc[b]], ssem[b], add=True)
        b3 = (b + DG) % NSLOT
        if b < 2:
          @pl.when(t > 0)
          def _():
            wait_scatter(b3)
          start_gather(k + DG, b3)
        else:
          wait_scatter(b3)

          @pl.when(t < TSEG // NSLOT - 1)
          def _():
            start_gather(k + DG, b3)

      return carry2

    lax.fori_loop(0, TSEG // NSLOT, iter_body, 0)
    wait_scatter(3)
    wait_scatter(4)
    return carry

  lax.fori_loop(0, NSEG, seg_body, 0)

  plsc.subcore_barrier()
  rbase = s * ROWS_PER_TILE
  pltpu.sync_copy(acc_sh.at[pl.ds(rbase, ROWS_PER_TILE)],
                  part_hbm.at[c, pl.ds(rbase, ROWS_PER_TILE)])


_sc_agg = functools.partial(
    pl.kernel,
    out_type=jax.ShapeDtypeStruct((2, NPAD, D), jnp.float32),
    mesh=plsc.VectorSubcoreMesh(core_axis_name="c", subcore_axis_name="s"),
    scratch_types=(
        [pltpu.VMEM((TBUF,), jnp.int32),
         pltpu.VMEM((TBUF,), jnp.int32),
         pltpu.VMEM((TBUF,), jnp.float32)]
        + [pltpu.VMEM((C, D), jnp.float32) for _ in range(NSLOT)]
        + [pltpu.VMEM((C,), jnp.int32) for _ in range(NSLOT)]
        + [pltpu.VMEM_SHARED((NPAD, D), jnp.float32)]
        + [pltpu.SemaphoreType.DMA for _ in range(2 * NSLOT)]
    ),
)(_sc_body)


BM = 1000


def _mm_body(p_ref, w_ref, o_ref):
  acc = p_ref[0] + p_ref[1]
  o_ref[...] = jnp.dot(acc, w_ref[...], preferred_element_type=jnp.float32)


def _combine_matmul(part, W):
  return pl.pallas_call(
      _mm_body,
      grid=(N // BM,),
      in_specs=[
          pl.BlockSpec((2, BM, D), lambda i: (0, i, 0)),
          pl.BlockSpec((D, D), lambda i: (0, 0)),
      ],
      out_specs=pl.BlockSpec((BM, D), lambda i: (i, 0)),
      out_shape=jax.ShapeDtypeStruct((N, D), jnp.float32),
  )(part, W)


@jax.jit
def kernel(x, edge_index, edge_weight, W):
  ipad = jnp.zeros((EPAD,), jnp.int32)
  dst = jnp.concatenate([edge_index[0].astype(jnp.int32), ipad])
  src = jnp.concatenate([edge_index[1].astype(jnp.int32), ipad])
  ew = jnp.concatenate([edge_weight, jnp.zeros((EPAD,), jnp.float32)])
  part = _sc_agg(x, src, dst, ew)
  return _combine_matmul(part, W)


# speedup vs baseline: 3.2942x; 3.2942x over previous
import functools

import jax
import jax.numpy as jnp
from jax import lax
from jax.experimental import pallas as pl
from jax.experimental.pallas import tpu as pltpu
from jax.experimental.pallas import tpu_sc as plsc

N = 10000
NPAD = 10240
D = 128
E = 320000
C = 64
NW = 32
NCHUNK = E // C
FULL_T = 160
ROWS_PER_TILE = NPAD // 16
TSEG = 20
NSEG = FULL_T // TSEG
TBUF = TSEG * C
NSLOT = 5
DG = 3


def _sc_body(x_hbm, src_hbm, dst_hbm, w_hbm, part_hbm,
             src_all, dst_all, w_all,
             rows0, rows1, rows2, rows3, rows4,
             dstc0, dstc1, dstc2, dstc3, dstc4, acc_sh,
             gsem0, gsem1, gsem2, gsem3, gsem4,
             ssem0, ssem1, ssem2, ssem3, ssem4):
  rows = [rows0, rows1, rows2, rows3, rows4]
  dstc = [dstc0, dstc1, dstc2, dstc3, dstc4]
  gsem = [gsem0, gsem1, gsem2, gsem3, gsem4]
  ssem = [ssem0, ssem1, ssem2, ssem3, ssem4]
  c = lax.axis_index("c")
  s = lax.axis_index("s")
  wid = s * 2 + c
  base = wid * FULL_T * C

  def zrow(i, carry):
    for j in range(8):
      rows0[i, pl.ds(j * 16, 16)] = jnp.zeros((16,), jnp.float32)
    return carry

  lax.fori_loop(0, C, zrow, 0)
  zd = []
  for r in range(ROWS_PER_TILE // C):
    zd.append(pltpu.async_copy(
        rows0, acc_sh.at[pl.ds(s * ROWS_PER_TILE + r * C, C)], ssem0))
  for dsc in zd:
    dsc.wait()

  plsc.subcore_barrier()

  def scale_chunk(buf, k):
    def scale16(i, carry):
      wv = w_all[pl.ds(k * C + i * 16, 16)]
      for r in range(16):
        wb = jnp.broadcast_to(wv[r], (16,))
        row = i * 16 + r
        for j in range(8):
          sl = pl.ds(j * 16, 16)
          buf[row, sl] = buf[row, sl] * wb
      return carry

    lax.fori_loop(0, C // 16, scale16, 0)

  def stage_dst(b, k):
    for j in range(C // 16):
      dstc[b][pl.ds(j * 16, 16)] = dst_all[pl.ds(k * C + j * 16, 16)]

  def start_gather(k, b):
    pltpu.async_copy(x_hbm.at[src_all.at[pl.ds(k * C, C)]], rows[b], gsem[b])

  def wait_gather(k, b):
    pltpu.make_async_copy(x_hbm.at[src_all.at[pl.ds(k * C, C)]], rows[b],
                          gsem[b]).wait()

  def wait_scatter(b):
    pltpu.make_async_copy(rows[b], acc_sh.at[dstc[b]], ssem[b]).wait()

  def seg_body(g, carry):
    sbase = base + g * TBUF
    pltpu.sync_copy(src_hbm.at[pl.ds(sbase, TBUF)], src_all)
    pltpu.sync_copy(dst_hbm.at[pl.ds(sbase, TBUF)], dst_all)
    pltpu.sync_copy(w_hbm.at[pl.ds(sbase, TBUF)], w_all)
    for j in range(DG):
      start_gather(j, j)

    def iter_body(t, carry2):
      for b in range(NSLOT):
        k = NSLOT * t + b
        wait_gather(k, b)
        scale_chunk(rows[b], k)
        stage_dst(b, k)
        pltpu.async_copy(rows[b], acc_sh.at[dstc[b]], ssem[b], add=True)
        b3 = (b + DG) % NSLOT
        if b < 2:
          @pl.when(t > 0)
          def _():
            wait_scatter(b3)
          start_gather(k + DG, b3)
        else:
          wait_scatter(b3)

          @pl.when(t < TSEG // NSLOT - 1)
          def _():
            start_gather(k + DG, b3)

      return carry2

    lax.fori_loop(0, TSEG // NSLOT, iter_body, 0)
    wait_scatter(3)
    wait_scatter(4)
    return carry

  nseg = jnp.where(wid < NW - 1, NSEG, (NCHUNK - (NW - 1) * FULL_T) // TSEG)
  lax.fori_loop(0, nseg, seg_body, 0)

  plsc.subcore_barrier()
  rbase = s * ROWS_PER_TILE
  pltpu.sync_copy(acc_sh.at[pl.ds(rbase, ROWS_PER_TILE)],
                  part_hbm.at[c, pl.ds(rbase, ROWS_PER_TILE)])


_sc_agg = functools.partial(
    pl.kernel,
    out_type=jax.ShapeDtypeStruct((2, NPAD, D), jnp.float32),
    mesh=plsc.VectorSubcoreMesh(core_axis_name="c", subcore_axis_name="s"),
    scratch_types=(
        [pltpu.VMEM((TBUF,), jnp.int32),
         pltpu.VMEM((TBUF,), jnp.int32),
         pltpu.VMEM((TBUF,), jnp.float32)]
        + [pltpu.VMEM((C, D), jnp.float32) for _ in range(NSLOT)]
        + [pltpu.VMEM((C,), jnp.int32) for _ in range(NSLOT)]
        + [pltpu.VMEM_SHARED((NPAD, D), jnp.float32)]
        + [pltpu.SemaphoreType.DMA for _ in range(2 * NSLOT)]
    ),
)(_sc_body)


BM = 1000


def _mm_body(p_ref, w_ref, o_ref):
  acc = p_ref[0] + p_ref[1]
  o_ref[...] = jnp.dot(acc, w_ref[...], preferred_element_type=jnp.float32)


def _combine_matmul(part, W):
  return pl.pallas_call(
      _mm_body,
      grid=(N // BM,),
      in_specs=[
          pl.BlockSpec((2, BM, D), lambda i: (0, i, 0)),
          pl.BlockSpec((D, D), lambda i: (0, 0)),
      ],
      out_specs=pl.BlockSpec((BM, D), lambda i: (i, 0)),
      out_shape=jax.ShapeDtypeStruct((N, D), jnp.float32),
  )(part, W)


@jax.jit
def kernel(x, edge_index, edge_weight, W):
  dst = edge_index[0].astype(jnp.int32)
  src = edge_index[1].astype(jnp.int32)
  part = _sc_agg(x, src, dst, edge_weight)
  return _combine_matmul(part, W)


# speedup vs baseline: 3.5277x; 1.0709x over previous
import functools

import jax
import jax.numpy as jnp
from jax import lax
from jax.experimental import pallas as pl
from jax.experimental.pallas import tpu as pltpu
from jax.experimental.pallas import tpu_sc as plsc

N = 10000
NPAD = 10240
D = 128
E = 320000
C = 64
NCHUNK = E // C
NW = 32
FULL_T = 156
REM = NCHUNK - FULL_T * NW
ROWS_PER_TILE = NPAD // 16
TSEG = 52
NSEG = FULL_T // TSEG
TBUF = TSEG * C
NSLOT = 4


def _sc_body(x_hbm, ei_hbm, w_hbm, part_hbm,
             src_all, dst_all, w_all,
             rows0, rows1, rows2, rows3, dstc0, dstc1, dstc2, dstc3, acc_sh,
             gsem0, gsem1, gsem2, gsem3, ssem0, ssem1, ssem2, ssem3):
  rows = [rows0, rows1, rows2, rows3]
  dstc = [dstc0, dstc1, dstc2, dstc3]
  gsem = [gsem0, gsem1, gsem2, gsem3]
  ssem = [ssem0, ssem1, ssem2, ssem3]
  c = lax.axis_index("c")
  s = lax.axis_index("s")
  wid = s * 2 + c
  base = (wid * FULL_T + jnp.minimum(wid, REM)) * C

  def zrow(i, carry):
    for j in range(8):
      rows0[i, pl.ds(j * 16, 16)] = jnp.zeros((16,), jnp.float32)
    return carry

  lax.fori_loop(0, C, zrow, 0)
  zd = []
  for r in range(ROWS_PER_TILE // C):
    zd.append(pltpu.async_copy(
        rows0, acc_sh.at[pl.ds(s * ROWS_PER_TILE + r * C, C)], ssem0))
  for dsc in zd:
    dsc.wait()

  plsc.subcore_barrier()

  def scale_chunk(buf, k):
    def scale16(i, carry):
      wv = w_all[pl.ds(k * C + i * 16, 16)]
      for r in range(16):
        wb = jnp.broadcast_to(wv[r], (16,))
        row = i * 16 + r
        for j in range(8):
          sl = pl.ds(j * 16, 16)
          buf[row, sl] = buf[row, sl] * wb
      return carry

    lax.fori_loop(0, C // 16, scale16, 0)

  def stage_dst(b, k):
    for j in range(C // 16):
      dstc[b][pl.ds(j * 16, 16)] = dst_all[pl.ds(k * C + j * 16, 16)]

  def start_gather(k, b):
    pltpu.async_copy(x_hbm.at[src_all.at[pl.ds(k * C, C)]], rows[b], gsem[b])

  def wait_gather(k, b):
    pltpu.make_async_copy(x_hbm.at[src_all.at[pl.ds(k * C, C)]], rows[b],
                          gsem[b]).wait()

  def wait_scatter(b):
    pltpu.make_async_copy(rows[b], acc_sh.at[dstc[b]], ssem[b]).wait()

  def seg_body(g, carry):
    sbase = base + g * TBUF
    pltpu.sync_copy(ei_hbm.at[pl.ds(E + sbase, TBUF)], src_all)
    pltpu.sync_copy(ei_hbm.at[pl.ds(sbase, TBUF)], dst_all)
    pltpu.sync_copy(w_hbm.at[pl.ds(sbase, TBUF)], w_all)
    start_gather(0, 0)
    start_gather(1, 1)

    def iter_body(t, carry2):
      for b in range(NSLOT):
        k = NSLOT * t + b
        wait_gather(k, b)
        scale_chunk(rows[b], k)
        stage_dst(b, k)
        pltpu.async_copy(rows[b], acc_sh.at[dstc[b]], ssem[b], add=True)
        b2 = (b + 2) % NSLOT
        if b < 2:
          @pl.when(t > 0)
          def _():
            wait_scatter(b2)
          start_gather(k + 2, b2)
        else:
          wait_scatter(b2)

          @pl.when(t < TSEG // NSLOT - 1)
          def _():
            start_gather(k + 2, b2)

      return carry2

    lax.fori_loop(0, TSEG // NSLOT, iter_body, 0)
    wait_scatter(2)
    wait_scatter(3)
    return carry

  lax.fori_loop(0, NSEG, seg_body, 0)

  @pl.when(wid < REM)
  def _():
    roff = base + FULL_T * C
    pltpu.sync_copy(ei_hbm.at[pl.ds(E + roff, C)], src_all.at[pl.ds(0, C)])
    pltpu.sync_copy(ei_hbm.at[pl.ds(roff, C)], dst_all.at[pl.ds(0, C)])
    pltpu.sync_copy(w_hbm.at[pl.ds(roff, C)], w_all.at[pl.ds(0, C)])
    pltpu.async_copy(x_hbm.at[src_all.at[pl.ds(0, C)]], rows0, gsem0).wait()
    scale_chunk(rows0, 0)
    stage_dst(0, 0)
    pltpu.sync_copy(rows0, acc_sh.at[dstc0], add=True)

  plsc.subcore_barrier()
  rbase = s * ROWS_PER_TILE
  pltpu.sync_copy(acc_sh.at[pl.ds(rbase, ROWS_PER_TILE)],
                  part_hbm.at[c, pl.ds(rbase, ROWS_PER_TILE)])


_sc_agg = functools.partial(
    pl.kernel,
    out_type=jax.ShapeDtypeStruct((2, NPAD, D), jnp.float32),
    mesh=plsc.VectorSubcoreMesh(core_axis_name="c", subcore_axis_name="s"),
    scratch_types=(
        [pltpu.VMEM((TBUF,), jnp.int32),
         pltpu.VMEM((TBUF,), jnp.int32),
         pltpu.VMEM((TBUF,), jnp.float32)]
        + [pltpu.VMEM((C, D), jnp.float32) for _ in range(NSLOT)]
        + [pltpu.VMEM((C,), jnp.int32) for _ in range(NSLOT)]
        + [pltpu.VMEM_SHARED((NPAD, D), jnp.float32)]
        + [pltpu.SemaphoreType.DMA for _ in range(2 * NSLOT)]
    ),
)(_sc_body)


BM = 1000


def _mm_body(p_ref, w_ref, o_ref):
  acc = p_ref[0] + p_ref[1]
  o_ref[...] = jnp.dot(acc, w_ref[...], preferred_element_type=jnp.float32)


def _combine_matmul(part, W):
  return pl.pallas_call(
      _mm_body,
      grid=(N // BM,),
      in_specs=[
          pl.BlockSpec((2, BM, D), lambda i: (0, i, 0)),
          pl.BlockSpec((D, D), lambda i: (0, 0)),
      ],
      out_specs=pl.BlockSpec((BM, D), lambda i: (i, 0)),
      out_shape=jax.ShapeDtypeStruct((N, D), jnp.float32),
  )(part, W)


@jax.jit
def kernel(x, edge_index, edge_weight, W):
  ei = edge_index.astype(jnp.int32).reshape(2 * E)
  part = _sc_agg(x, ei, edge_weight)
  return _combine_matmul(part, W)
